# baseline (speedup 1.0000x reference)
import functools

import jax
import jax.numpy as jnp
from jax import lax
from jax.experimental import pallas as pl
from jax.experimental.pallas import tpu as pltpu
from jax.experimental.pallas import tpu_sc as plsc

E = 160000
NNEI = 8
IN = 128
HID = 64
F32 = jnp.float32
BF16 = jnp.bfloat16
NEG_SCALE = -1.0

BE = 2000
NC = 2
NS = 16
NW = NC * NS
PER_W = E // NW
CH = 8
NCHUNK = PER_W // CH
ROWS = CH * NNEI
NBUF = 2
LANES = 16
NV = HID // LANES
REC = 128


def _sig(x):
    return 1.0 / (1.0 + jnp.exp(-x))


def _keep_mask(nrows):
    pid = pl.program_id(0)
    rowids = lax.broadcasted_iota(jnp.int32, (nrows, 1), 0) + pid * nrows
    return (rowids != 0).astype(F32)


def _bf16_word(a, b):
    ai = lax.bitcast_convert_type(a.astype(BF16).astype(F32), jnp.int32)
    bi = lax.bitcast_convert_type(b.astype(BF16).astype(F32), jnp.int32)
    return lax.bitwise_or(lax.shift_right_logical(ai, 16),
                          lax.bitwise_and(bi, jnp.int32(-65536)))


def _pack_rec(h, c, wfh_ref, rec_ref):
    g = jnp.dot(h, wfh_ref[...], preferred_element_type=F32)
    rec_ref[...] = jnp.concatenate(
        [_bf16_word(h[:, :32], h[:, 32:]), _bf16_word(g, c),
         jnp.zeros((BE, 32), jnp.int32)], axis=1)


def _phase_a_body(f_ref, wx_ref, b_ref, wfh_ref, pio_ref, pf_ref, rec_ref):
    pre = jnp.dot(f_ref[...], wx_ref[...], preferred_element_type=F32) + b_ref[...]
    pio_ref[...] = pre[:, :3 * HID]
    pf_ref[...] = pre[:, 3 * HID:]
    i0 = _sig(pre[:, :HID])
    o0 = _sig(pre[:, HID:2 * HID])
    u0 = jnp.tanh(pre[:, 2 * HID:3 * HID])
    keep = _keep_mask(BE)
    c1 = i0 * u0 * keep
    h1 = o0 * jnp.tanh(c1) * keep
    _pack_rec(h1, c1, wfh_ref, rec_ref)


def _phase_c_mid_body(pio_ref, hs_ref, ca_ref, wh_ref, wfh_ref, rec_ref):
    t = pio_ref[...] + jnp.dot(hs_ref[...], wh_ref[...], preferred_element_type=F32)
    i = _sig(t[:, :HID])
    o = _sig(t[:, HID:2 * HID])
    u = jnp.tanh(t[:, 2 * HID:])
    keep = _keep_mask(BE)
    c = (i * u + ca_ref[...]) * keep
    h = o * jnp.tanh(c) * keep
    _pack_rec(h, c, wfh_ref, rec_ref)


def _phase_c_final_body(pio_ref, hs_ref, ca_ref, wh_ref, h_ref, c_ref):
    t = pio_ref[...] + jnp.dot(hs_ref[...], wh_ref[...], preferred_element_type=F32)
    i = _sig(t[:, :HID])
    o = _sig(t[:, HID:2 * HID])
    u = jnp.tanh(t[:, 2 * HID:])
    keep = _keep_mask(BE)
    c = (i * u + ca_ref[...]) * keep
    h = o * jnp.tanh(c) * keep
    h_ref[...] = h
    c_ref[...] = c


def _deint(xi):
    ev = plsc.bitcast(lax.shift_left(xi, 16), F32)
    od = plsc.bitcast(lax.bitwise_and(xi, jnp.int32(-65536)), F32)
    return ev, od


def _sc_body(rec_hbm, pref_hbm, idx_hbm, hs_hbm, ca_hbm,
             idx_all, rows0, rows1, pref0, pref1,
             hs0, hs1, ca0, ca1,
             gsem0, gsem1, psem0, psem1, ssem0, ssem1):
    wid = lax.axis_index("s") * NC + lax.axis_index("c")
    base = wid * PER_W
    rows = (rows0, rows1)
    pref = (pref0, pref1)
    hsb = (hs0, hs1)
    cab = (ca0, ca1)
    gsem = (gsem0, gsem1)
    psem = (psem0, psem1)
    ssem = (ssem0, ssem1)

    def g_copy(i, b):
        return pltpu.make_async_copy(
            rec_hbm.at[idx_all.at[pl.ds(i * ROWS, ROWS)]], rows[b], gsem[b])

    def p_copy(i, b):
        return pltpu.make_async_copy(
            pref_hbm.at[pl.ds(base + i * CH, CH)], pref[b], psem[b])

    def s_copies(i, b):
        m0 = base + i * CH
        return (pltpu.make_async_copy(hsb[b], hs_hbm.at[pl.ds(m0, CH)], ssem[b]),
                pltpu.make_async_copy(cab[b], ca_hbm.at[pl.ds(m0, CH)], ssem[b]))

    def compute(b):
        @plsc.parallel_loop(0, CH, unroll=2)
        def msg_body(m):
            r0 = m * NNEI
            p = [pref[b][m, pl.ds(v * LANES, LANES)] for v in range(NV)]
            hacc = [None, None]
            cacc = [None] * NV
            for k in range(NNEI):
                r = r0 + k
                for g in range(2):
                    hv = plsc.bitcast(rows[b][r, pl.ds(g * 16, 16)], BF16)
                    hacc[g] = hv if k == 0 else hacc[g] + hv
                for v in range(NV):
                    gq, cq = _deint(rows[b][r, pl.ds(32 + 16 * v, 16)])
                    f = cq / (1.0 + jnp.exp(p[v] + gq))
                    cacc[v] = f if k == 0 else cacc[v] + f
            for g in range(2):
                ev, od = _deint(plsc.bitcast(hacc[g], jnp.int32))
                hsb[b][m, pl.ds((2 * g) * LANES, LANES)] = ev
                hsb[b][m, pl.ds((2 * g + 1) * LANES, LANES)] = od
            for v in range(NV):
                cab[b][m, pl.ds(v * LANES, LANES)] = cacc[v]

    pltpu.sync_copy(idx_hbm.at[pl.ds(base * NNEI, PER_W * NNEI)], idx_all)
    for b in range(NBUF):
        g_copy(b, b).start()
        p_copy(b, b).start()

    def group_body(pp, carry):
        for b in range(NBUF):
            i = pp * NBUF + b
            g_copy(i, b).wait()
            p_copy(i, b).wait()

            @pl.when(pp > 0)
            def _wait_store():
                sh, sc = s_copies(i, b)
                sh.wait()
                sc.wait()

            compute(b)
            sh, sc = s_copies(i, b)
            sh.start()
            sc.start()

            @pl.when(i + NBUF < NCHUNK)
            def _prefetch():
                g_copy(i + NBUF, b).start()
                p_copy(i + NBUF, b).start()
        return carry

    ngroups = NCHUNK // NBUF
    lax.fori_loop(0, ngroups, group_body, 0)
    for t in range(ngroups * NBUF, NCHUNK):
        b = t % NBUF
        g_copy(t, b).wait()
        p_copy(t, b).wait()
        sh, sc = s_copies(t, b)
        sh.wait()
        sc.wait()
        compute(b)
        sh, sc = s_copies(t, b)
        sh.start()
        sc.start()
    for b in range(NBUF):
        sh, sc = s_copies(NCHUNK - 1, b)
        sh.wait()
        sc.wait()


@functools.cache
def _sc_gather():
    mesh = plsc.VectorSubcoreMesh(core_axis_name="c", subcore_axis_name="s")
    return pl.kernel(
        _sc_body,
        mesh=mesh,
        compiler_params=pltpu.CompilerParams(needs_layout_passes=False),
        out_type=(jax.ShapeDtypeStruct((E, HID), F32),
                  jax.ShapeDtypeStruct((E, HID), F32)),
        scratch_types=(
            [pltpu.VMEM((PER_W * NNEI,), jnp.int32)]
            + [pltpu.VMEM((ROWS, 128), jnp.int32) for _ in range(NBUF)]
            + [pltpu.VMEM((CH, HID), F32) for _ in range(3 * NBUF)]
            + [pltpu.SemaphoreType.DMA for _ in range(3 * NBUF)]
        ),
    )


def _rows(w):
    return pl.BlockSpec((BE, w), lambda i: (i, 0))


def _full(r, c):
    return pl.BlockSpec((r, c), lambda i: (0, 0))


def kernel(fmess, bgraph, Wi, bi, Wo, bo, Wf, bf, Wu, bu):
    s = jnp.float32(NEG_SCALE)
    Wx = jnp.concatenate(
        [Wi[:, :IN], Wo[:, :IN], Wu[:, :IN], Wf[:, :IN] * s], axis=0).T
    bcat = jnp.concatenate([bi, bo, bu, bf * s]).reshape(1, 4 * HID)
    Wh = jnp.concatenate([Wi[:, IN:], Wo[:, IN:], Wu[:, IN:]], axis=0).T
    q = jnp.concatenate([jnp.arange(0, 16), jnp.arange(32, 48),
                         jnp.arange(16, 32), jnp.arange(48, 64)])
    Wh = Wh[q]
    WfhT = Wf[:, IN:].T * s
    idx = bgraph.reshape(-1)

    grid = (E // BE,)
    params = pltpu.CompilerParams(dimension_semantics=("parallel",))

    pio, pf, rec = pl.pallas_call(
        _phase_a_body,
        grid=grid,
        in_specs=[_rows(IN), _full(IN, 4 * HID), _full(1, 4 * HID),
                  _full(HID, HID)],
        out_specs=[_rows(3 * HID), _rows(HID), _rows(REC)],
        out_shape=[jax.ShapeDtypeStruct((E, 3 * HID), F32),
                   jax.ShapeDtypeStruct((E, HID), F32),
                   jax.ShapeDtypeStruct((E, REC), jnp.int32)],
        compiler_params=params,
    )(fmess, Wx, bcat, WfhT)

    scg = _sc_gather()
    hs, ca = scg(rec, pf, idx)

    rec = pl.pallas_call(
        _phase_c_mid_body,
        grid=grid,
        in_specs=[_rows(3 * HID), _rows(HID), _rows(HID),
                  _full(HID, 3 * HID), _full(HID, HID)],
        out_specs=_rows(REC),
        out_shape=jax.ShapeDtypeStruct((E, REC), jnp.int32),
        compiler_params=params,
    )(pio, hs, ca, Wh, WfhT)

    hs, ca = scg(rec, pf, idx)

    h, c = pl.pallas_call(
        _phase_c_final_body,
        grid=grid,
        in_specs=[_rows(3 * HID), _rows(HID), _rows(HID), _full(HID, 3 * HID)],
        out_specs=[_rows(HID), _rows(HID)],
        out_shape=[jax.ShapeDtypeStruct((E, HID), F32),
                   jax.ShapeDtypeStruct((E, HID), F32)],
        compiler_params=params,
    )(pio, hs, ca, Wh)

    return (h, c)


# speedup vs baseline: 1.6953x; 1.6953x over previous
import functools

import jax
import jax.numpy as jnp
from jax import lax
from jax.experimental import pallas as pl
from jax.experimental.pallas import tpu as pltpu
from jax.experimental.pallas import tpu_sc as plsc

E = 160000
NNEI = 8
IN = 128
HID = 64
F32 = jnp.float32
BF16 = jnp.bfloat16
NEG_SCALE = -1.0

BE = 2000
NC = 2
NS = 16
NW = NC * NS
PER_W = E // NW
CH = 8
NCHUNK = PER_W // CH
ROWS = CH * NNEI
NBUF = 2
LANES = 16
NV = HID // LANES
REC = 128


def _sig(x):
    return 1.0 / (1.0 + jnp.exp(-x))


def _keep_mask(nrows):
    pid = pl.program_id(0)
    rowids = lax.broadcasted_iota(jnp.int32, (nrows, 1), 0) + pid * nrows
    return (rowids != 0).astype(F32)


def _bf16_word(a, b):
    ai = lax.bitcast_convert_type(a.astype(BF16).astype(F32), jnp.int32)
    bi = lax.bitcast_convert_type(b.astype(BF16).astype(F32), jnp.int32)
    return lax.bitwise_or(lax.shift_right_logical(ai, 16),
                          lax.bitwise_and(bi, jnp.int32(-65536)))


def _pack_rec(h, c, wfh_ref, rec_ref):
    g = jnp.dot(h, wfh_ref[...], preferred_element_type=F32)
    rec_ref[...] = jnp.concatenate(
        [_bf16_word(h[:, :32], h[:, 32:]), _bf16_word(g, c),
         jnp.zeros((BE, 32), jnp.int32)], axis=1)


def _phase_a_body(f_ref, wx_ref, b_ref, wfh_ref, pio_ref, pf_ref, rec_ref):
    pre = jnp.dot(f_ref[...], wx_ref[...], preferred_element_type=F32) + b_ref[...]
    pio_ref[...] = pre[:, :3 * HID]
    pf_ref[...] = pre[:, 3 * HID:]
    i0 = _sig(pre[:, :HID])
    o0 = _sig(pre[:, HID:2 * HID])
    u0 = jnp.tanh(pre[:, 2 * HID:3 * HID])
    keep = _keep_mask(BE)
    c1 = i0 * u0 * keep
    h1 = o0 * jnp.tanh(c1) * keep
    _pack_rec(h1, c1, wfh_ref, rec_ref)


def _phase_c_mid_body(pio_ref, hs_ref, ca_ref, wh_ref, wfh_ref, rec_ref):
    t = pio_ref[...] + jnp.dot(hs_ref[...], wh_ref[...], preferred_element_type=F32)
    i = _sig(t[:, :HID])
    o = _sig(t[:, HID:2 * HID])
    u = jnp.tanh(t[:, 2 * HID:])
    keep = _keep_mask(BE)
    c = (i * u + ca_ref[...]) * keep
    h = o * jnp.tanh(c) * keep
    _pack_rec(h, c, wfh_ref, rec_ref)


def _phase_c_final_body(pio_ref, hs_ref, ca_ref, wh_ref, h_ref, c_ref):
    t = pio_ref[...] + jnp.dot(hs_ref[...], wh_ref[...], preferred_element_type=F32)
    i = _sig(t[:, :HID])
    o = _sig(t[:, HID:2 * HID])
    u = jnp.tanh(t[:, 2 * HID:])
    keep = _keep_mask(BE)
    c = (i * u + ca_ref[...]) * keep
    h = o * jnp.tanh(c) * keep
    h_ref[...] = h
    c_ref[...] = c


def _deint(xi):
    ev = plsc.bitcast(lax.shift_left(xi, 16), F32)
    od = plsc.bitcast(lax.bitwise_and(xi, jnp.int32(-65536)), F32)
    return ev, od


def _sc_body(rec_hbm, pref_hbm, idx_hbm, hs_hbm, ca_hbm,
             idx_all, rows0, rows1, pref0, pref1,
             hs0, hs1, ca0, ca1,
             gsem0, gsem1, psem0, psem1, ssem0, ssem1):
    wid = lax.axis_index("s") * NC + lax.axis_index("c")
    base = wid * PER_W
    rows = (rows0, rows1)
    pref = (pref0, pref1)
    hsb = (hs0, hs1)
    cab = (ca0, ca1)
    gsem = (gsem0, gsem1)
    psem = (psem0, psem1)
    ssem = (ssem0, ssem1)

    def g_copy(i, b):
        return pltpu.make_async_copy(
            rec_hbm.at[idx_all.at[pl.ds(i * ROWS, ROWS)]], rows[b], gsem[b])

    def p_copy(i, b):
        return pltpu.make_async_copy(
            pref_hbm.at[pl.ds(base + i * CH, CH)], pref[b], psem[b])

    def s_copies(i, b):
        m0 = base + i * CH
        return (pltpu.make_async_copy(hsb[b], hs_hbm.at[pl.ds(m0, CH)], ssem[b]),
                pltpu.make_async_copy(cab[b], ca_hbm.at[pl.ds(m0, CH)], ssem[b]))

    def compute(b):
        @plsc.parallel_loop(0, CH, unroll=2)
        def msg_body(m):
            r0 = m * NNEI
            p = [pref[b][m, pl.ds(v * LANES, LANES)] for v in range(NV)]
            hacc = [None] * NV
            cacc = [None] * NV
            for k in range(NNEI):
                r = r0 + k
                for g in range(2):
                    ev, od = _deint(rows[b][r, pl.ds(g * 16, 16)])
                    hacc[2 * g] = ev if k == 0 else hacc[2 * g] + ev
                    hacc[2 * g + 1] = od if k == 0 else hacc[2 * g + 1] + od
                for v in range(NV):
                    gq, cq = _deint(rows[b][r, pl.ds(32 + 16 * v, 16)])
                    f = cq / (1.0 + jnp.exp(p[v] + gq))
                    cacc[v] = f if k == 0 else cacc[v] + f
            for v in range(NV):
                hsb[b][m, pl.ds(v * LANES, LANES)] = hacc[v]
                cab[b][m, pl.ds(v * LANES, LANES)] = cacc[v]

    pltpu.sync_copy(idx_hbm.at[pl.ds(base * NNEI, PER_W * NNEI)], idx_all)
    for b in range(NBUF):
        g_copy(b, b).start()
        p_copy(b, b).start()

    def group_body(pp, carry):
        for b in range(NBUF):
            i = pp * NBUF + b
            g_copy(i, b).wait()
            p_copy(i, b).wait()

            @pl.when(pp > 0)
            def _wait_store():
                sh, sc = s_copies(i, b)
                sh.wait()
                sc.wait()

            compute(b)
            sh, sc = s_copies(i, b)
            sh.start()
            sc.start()

            @pl.when(i + NBUF < NCHUNK)
            def _prefetch():
                g_copy(i + NBUF, b).start()
                p_copy(i + NBUF, b).start()
        return carry

    ngroups = NCHUNK // NBUF
    lax.fori_loop(0, ngroups, group_body, 0)
    for t in range(ngroups * NBUF, NCHUNK):
        b = t % NBUF
        g_copy(t, b).wait()
        p_copy(t, b).wait()
        sh, sc = s_copies(t, b)
        sh.wait()
        sc.wait()
        compute(b)
        sh, sc = s_copies(t, b)
        sh.start()
        sc.start()
    for b in range(NBUF):
        sh, sc = s_copies(NCHUNK - 1, b)
        sh.wait()
        sc.wait()


@functools.cache
def _sc_gather():
    mesh = plsc.VectorSubcoreMesh(core_axis_name="c", subcore_axis_name="s")
    return pl.kernel(
        _sc_body,
        mesh=mesh,
        compiler_params=pltpu.CompilerParams(needs_layout_passes=False,
                                             use_tc_tiling_on_sc=True),
        out_type=(jax.ShapeDtypeStruct((E, HID), F32),
                  jax.ShapeDtypeStruct((E, HID), F32)),
        scratch_types=(
            [pltpu.VMEM((PER_W * NNEI,), jnp.int32)]
            + [pltpu.VMEM((ROWS, 128), jnp.int32) for _ in range(NBUF)]
            + [pltpu.VMEM((CH, HID), F32) for _ in range(3 * NBUF)]
            + [pltpu.SemaphoreType.DMA for _ in range(3 * NBUF)]
        ),
    )


def _rows(w):
    return pl.BlockSpec((BE, w), lambda i: (i, 0))


def _full(r, c):
    return pl.BlockSpec((r, c), lambda i: (0, 0))


def kernel(fmess, bgraph, Wi, bi, Wo, bo, Wf, bf, Wu, bu):
    s = jnp.float32(NEG_SCALE)
    Wx = jnp.concatenate(
        [Wi[:, :IN], Wo[:, :IN], Wu[:, :IN], Wf[:, :IN] * s], axis=0).T
    bcat = jnp.concatenate([bi, bo, bu, bf * s]).reshape(1, 4 * HID)
    Wh = jnp.concatenate([Wi[:, IN:], Wo[:, IN:], Wu[:, IN:]], axis=0).T
    q = jnp.concatenate([jnp.arange(0, 16), jnp.arange(32, 48),
                         jnp.arange(16, 32), jnp.arange(48, 64)])
    Wh = Wh[q]
    WfhT = Wf[:, IN:].T * s
    idx = bgraph.reshape(-1)

    grid = (E // BE,)
    params = pltpu.CompilerParams(dimension_semantics=("parallel",))

    pio, pf, rec = pl.pallas_call(
        _phase_a_body,
        grid=grid,
        in_specs=[_rows(IN), _full(IN, 4 * HID), _full(1, 4 * HID),
                  _full(HID, HID)],
        out_specs=[_rows(3 * HID), _rows(HID), _rows(REC)],
        out_shape=[jax.ShapeDtypeStruct((E, 3 * HID), F32),
                   jax.ShapeDtypeStruct((E, HID), F32),
                   jax.ShapeDtypeStruct((E, REC), jnp.int32)],
        compiler_params=params,
    )(fmess, Wx, bcat, WfhT)

    scg = _sc_gather()
    hs, ca = scg(rec, pf, idx)

    rec = pl.pallas_call(
        _phase_c_mid_body,
        grid=grid,
        in_specs=[_rows(3 * HID), _rows(HID), _rows(HID),
                  _full(HID, 3 * HID), _full(HID, HID)],
        out_specs=_rows(REC),
        out_shape=jax.ShapeDtypeStruct((E, REC), jnp.int32),
        compiler_params=params,
    )(pio, hs, ca, Wh, WfhT)

    hs, ca = scg(rec, pf, idx)

    h, c = pl.pallas_call(
        _phase_c_final_body,
        grid=grid,
        in_specs=[_rows(3 * HID), _rows(HID), _rows(HID), _full(HID, 3 * HID)],
        out_specs=[_rows(HID), _rows(HID)],
        out_shape=[jax.ShapeDtypeStruct((E, HID), F32),
                   jax.ShapeDtypeStruct((E, HID), F32)],
        compiler_params=params,
    )(pio, hs, ca, Wh)

    return (h, c)


# speedup vs baseline: 1.7319x; 1.0216x over previous
import functools

import jax
import jax.numpy as jnp
from jax import lax
from jax.experimental import pallas as pl
from jax.experimental.pallas import tpu as pltpu
from jax.experimental.pallas import tpu_sc as plsc

E = 160000
NNEI = 8
IN = 128
HID = 64
F32 = jnp.float32
BF16 = jnp.bfloat16
NEG_SCALE = -1.0

BE = 2000
NC = 2
NS = 16
NW = NC * NS
PER_W = E // NW
CH = 8
NCHUNK = PER_W // CH
ROWS = CH * NNEI
NBUF = 2
LANES = 16
NV = HID // LANES
REC = 128


def _sig(x):
    return 1.0 / (1.0 + jnp.exp(-x))


def _keep_mask(nrows):
    pid = pl.program_id(0)
    rowids = lax.broadcasted_iota(jnp.int32, (nrows, 1), 0) + pid * nrows
    return (rowids != 0).astype(F32)


def _bf16_word(a, b):
    ai = lax.bitcast_convert_type(a.astype(BF16).astype(F32), jnp.int32)
    bi = lax.bitcast_convert_type(b.astype(BF16).astype(F32), jnp.int32)
    return lax.bitwise_or(lax.shift_right_logical(ai, 16),
                          lax.bitwise_and(bi, jnp.int32(-65536)))


def _pack_rec(h, c, wfh_ref, rec_ref):
    g = jnp.dot(h, wfh_ref[...], preferred_element_type=F32)
    rec_ref[...] = jnp.concatenate(
        [_bf16_word(h[:, :32], h[:, 32:]), _bf16_word(g, c),
         jnp.zeros((BE, 32), jnp.int32)], axis=1)


def _phase_a_body(f_ref, wx_ref, b_ref, wfh_ref, pf_ref, rec_ref):
    pre = jnp.dot(f_ref[...], wx_ref[...], preferred_element_type=F32) + b_ref[...]
    pf_ref[...] = pre[:, 3 * HID:]
    i0 = _sig(pre[:, :HID])
    o0 = _sig(pre[:, HID:2 * HID])
    u0 = jnp.tanh(pre[:, 2 * HID:3 * HID])
    keep = _keep_mask(BE)
    c1 = i0 * u0 * keep
    h1 = o0 * jnp.tanh(c1) * keep
    _pack_rec(h1, c1, wfh_ref, rec_ref)


def _phase_c_mid_body(f_ref, wx2_ref, b2_ref, hs_ref, ca_ref, wh_ref, wfh_ref,
                      rec_ref):
    t = (jnp.dot(f_ref[...], wx2_ref[...], preferred_element_type=F32)
         + jnp.dot(hs_ref[...], wh_ref[...], preferred_element_type=F32)
         + b2_ref[...])
    i = _sig(t[:, :HID])
    o = _sig(t[:, HID:2 * HID])
    u = jnp.tanh(t[:, 2 * HID:])
    keep = _keep_mask(BE)
    c = (i * u + ca_ref[...]) * keep
    h = o * jnp.tanh(c) * keep
    _pack_rec(h, c, wfh_ref, rec_ref)


def _phase_c_final_body(f_ref, wx2_ref, b2_ref, hs_ref, ca_ref, wh_ref,
                        h_ref, c_ref):
    t = (jnp.dot(f_ref[...], wx2_ref[...], preferred_element_type=F32)
         + jnp.dot(hs_ref[...], wh_ref[...], preferred_element_type=F32)
         + b2_ref[...])
    i = _sig(t[:, :HID])
    o = _sig(t[:, HID:2 * HID])
    u = jnp.tanh(t[:, 2 * HID:])
    keep = _keep_mask(BE)
    c = (i * u + ca_ref[...]) * keep
    h = o * jnp.tanh(c) * keep
    h_ref[...] = h
    c_ref[...] = c


def _deint(xi):
    ev = plsc.bitcast(lax.shift_left(xi, 16), F32)
    od = plsc.bitcast(lax.bitwise_and(xi, jnp.int32(-65536)), F32)
    return ev, od


def _sc_body(rec_hbm, pref_hbm, idx_hbm, hs_hbm, ca_hbm,
             idx_all, rows0, rows1, pref0, pref1,
             hs0, hs1, ca0, ca1,
             gsem0, gsem1, psem0, psem1, ssem0, ssem1):
    wid = lax.axis_index("s") * NC + lax.axis_index("c")
    base = wid * PER_W
    rows = (rows0, rows1)
    pref = (pref0, pref1)
    hsb = (hs0, hs1)
    cab = (ca0, ca1)
    gsem = (gsem0, gsem1)
    psem = (psem0, psem1)
    ssem = (ssem0, ssem1)

    def g_copy(i, b):
        return pltpu.make_async_copy(
            rec_hbm.at[idx_all.at[pl.ds(i * ROWS, ROWS)]], rows[b], gsem[b])

    def p_copy(i, b):
        return pltpu.make_async_copy(
            pref_hbm.at[pl.ds(base + i * CH, CH)], pref[b], psem[b])

    def s_copies(i, b):
        m0 = base + i * CH
        return (pltpu.make_async_copy(hsb[b], hs_hbm.at[pl.ds(m0, CH)], ssem[b]),
                pltpu.make_async_copy(cab[b], ca_hbm.at[pl.ds(m0, CH)], ssem[b]))

    def compute(b):
        @plsc.parallel_loop(0, CH, unroll=2)
        def msg_body(m):
            r0 = m * NNEI
            p = [pref[b][m, pl.ds(v * LANES, LANES)] for v in range(NV)]
            hacc = [None] * NV
            cacc = [None] * NV
            for k in range(NNEI):
                r = r0 + k
                for g in range(2):
                    ev, od = _deint(rows[b][r, pl.ds(g * 16, 16)])
                    hacc[2 * g] = ev if k == 0 else hacc[2 * g] + ev
                    hacc[2 * g + 1] = od if k == 0 else hacc[2 * g + 1] + od
                for v in range(NV):
                    gq, cq = _deint(rows[b][r, pl.ds(32 + 16 * v, 16)])
                    f = cq / (1.0 + jnp.exp(p[v] + gq))
                    cacc[v] = f if k == 0 else cacc[v] + f
            for v in range(NV):
                hsb[b][m, pl.ds(v * LANES, LANES)] = hacc[v]
                cab[b][m, pl.ds(v * LANES, LANES)] = cacc[v]

    pltpu.sync_copy(idx_hbm.at[pl.ds(base * NNEI, PER_W * NNEI)], idx_all)
    for b in range(NBUF):
        g_copy(b, b).start()
        p_copy(b, b).start()

    def group_body(pp, carry):
        for b in range(NBUF):
            i = pp * NBUF + b
            g_copy(i, b).wait()
            p_copy(i, b).wait()

            @pl.when(pp > 0)
            def _wait_store():
                sh, sc = s_copies(i, b)
                sh.wait()
                sc.wait()

            compute(b)
            sh, sc = s_copies(i, b)
            sh.start()
            sc.start()

            @pl.when(i + NBUF < NCHUNK)
            def _prefetch():
                g_copy(i + NBUF, b).start()
                p_copy(i + NBUF, b).start()
        return carry

    ngroups = NCHUNK // NBUF
    lax.fori_loop(0, ngroups, group_body, 0)
    for t in range(ngroups * NBUF, NCHUNK):
        b = t % NBUF
        g_copy(t, b).wait()
        p_copy(t, b).wait()
        sh, sc = s_copies(t, b)
        sh.wait()
        sc.wait()
        compute(b)
        sh, sc = s_copies(t, b)
        sh.start()
        sc.start()
    for b in range(NBUF):
        sh, sc = s_copies(NCHUNK - 1, b)
        sh.wait()
        sc.wait()


@functools.cache
def _sc_gather():
    mesh = plsc.VectorSubcoreMesh(core_axis_name="c", subcore_axis_name="s")
    return pl.kernel(
        _sc_body,
        mesh=mesh,
        compiler_params=pltpu.CompilerParams(needs_layout_passes=False,
                                             use_tc_tiling_on_sc=True),
        out_type=(jax.ShapeDtypeStruct((E, HID), F32),
                  jax.ShapeDtypeStruct((E, HID), F32)),
        scratch_types=(
            [pltpu.VMEM((PER_W * NNEI,), jnp.int32)]
            + [pltpu.VMEM((ROWS, 128), jnp.int32) for _ in range(NBUF)]
            + [pltpu.VMEM((CH, HID), F32) for _ in range(3 * NBUF)]
            + [pltpu.SemaphoreType.DMA for _ in range(3 * NBUF)]
        ),
    )


def _rows(w):
    return pl.BlockSpec((BE, w), lambda i: (i, 0))


def _full(r, c):
    return pl.BlockSpec((r, c), lambda i: (0, 0))


def kernel(fmess, bgraph, Wi, bi, Wo, bo, Wf, bf, Wu, bu):
    s = jnp.float32(NEG_SCALE)
    Wx = jnp.concatenate(
        [Wi[:, :IN], Wo[:, :IN], Wu[:, :IN], Wf[:, :IN] * s], axis=0).T
    bcat = jnp.concatenate([bi, bo, bu, bf * s]).reshape(1, 4 * HID)
    Wh = jnp.concatenate([Wi[:, IN:], Wo[:, IN:], Wu[:, IN:]], axis=0).T
    q = jnp.concatenate([jnp.arange(0, 16), jnp.arange(32, 48),
                         jnp.arange(16, 32), jnp.arange(48, 64)])
    Wh = Wh[q]
    WfhT = Wf[:, IN:].T * s
    Wx2 = Wx[:, :3 * HID]
    b2 = bcat[:, :3 * HID]
    idx = bgraph.reshape(-1)

    grid = (E // BE,)
    params = pltpu.CompilerParams(dimension_semantics=("parallel",))

    pf, rec = pl.pallas_call(
        _phase_a_body,
        grid=grid,
        in_specs=[_rows(IN), _full(IN, 4 * HID), _full(1, 4 * HID),
                  _full(HID, HID)],
        out_specs=[_rows(HID), _rows(REC)],
        out_shape=[jax.ShapeDtypeStruct((E, HID), F32),
                   jax.ShapeDtypeStruct((E, REC), jnp.int32)],
        compiler_params=params,
    )(fmess, Wx, bcat, WfhT)

    scg = _sc_gather()
    hs, ca = scg(rec, pf, idx)

    rec = pl.pallas_call(
        _phase_c_mid_body,
        grid=grid,
        in_specs=[_rows(IN), _full(IN, 3 * HID), _full(1, 3 * HID),
                  _rows(HID), _rows(HID), _full(HID, 3 * HID), _full(HID, HID)],
        out_specs=_rows(REC),
        out_shape=jax.ShapeDtypeStruct((E, REC), jnp.int32),
        compiler_params=params,
    )(fmess, Wx2, b2, hs, ca, Wh, WfhT)

    hs, ca = scg(rec, pf, idx)

    h, c = pl.pallas_call(
        _phase_c_final_body,
        grid=grid,
        in_specs=[_rows(IN), _full(IN, 3 * HID), _full(1, 3 * HID),
                  _rows(HID), _rows(HID), _full(HID, 3 * HID)],
        out_specs=[_rows(HID), _rows(HID)],
        out_shape=[jax.ShapeDtypeStruct((E, HID), F32),
                   jax.ShapeDtypeStruct((E, HID), F32)],
        compiler_params=params,
    )(fmess, Wx2, b2, hs, ca, Wh)

    return (h, c)


# speedup vs baseline: 1.8237x; 1.0530x over previous
import functools

import jax
import jax.numpy as jnp
from jax import lax
from jax.experimental import pallas as pl
from jax.experimental.pallas import tpu as pltpu
from jax.experimental.pallas import tpu_sc as plsc

E = 160000
NNEI = 8
IN = 128
HID = 64
F32 = jnp.float32
BF16 = jnp.bfloat16
NEG_SCALE = -1.0

BE = 2000
NC = 2
NS = 16
NW = NC * NS
PER_W = E // NW
CH = 16
NCHUNK = PER_W // CH
TAIL = PER_W - NCHUNK * CH
ROWS = CH * NNEI
NBUF = 2
LANES = 16
NV = HID // LANES
REC = 128


def _sig(x):
    return 1.0 / (1.0 + jnp.exp(-x))


def _keep_mask(nrows):
    pid = pl.program_id(0)
    rowids = lax.broadcasted_iota(jnp.int32, (nrows, 1), 0) + pid * nrows
    return (rowids != 0).astype(F32)


def _bf16_word(a, b):
    ai = lax.bitcast_convert_type(a.astype(BF16).astype(F32), jnp.int32)
    bi = lax.bitcast_convert_type(b.astype(BF16).astype(F32), jnp.int32)
    return lax.bitwise_or(lax.shift_right_logical(ai, 16),
                          lax.bitwise_and(bi, jnp.int32(-65536)))


def _pack_rec(h, c, wfh_ref, rec_ref):
    g = jnp.dot(h, wfh_ref[...], preferred_element_type=F32)
    rec_ref[...] = jnp.concatenate(
        [_bf16_word(h[:, :32], h[:, 32:]), _bf16_word(g, c),
         jnp.zeros((BE, 32), jnp.int32)], axis=1)


def _phase_a_body(f_ref, wx_ref, b_ref, wfh_ref, pf_ref, rec_ref):
    pre = jnp.dot(f_ref[...], wx_ref[...], preferred_element_type=F32) + b_ref[...]
    pf_ref[...] = pre[:, 3 * HID:]
    i0 = _sig(pre[:, :HID])
    o0 = _sig(pre[:, HID:2 * HID])
    u0 = jnp.tanh(pre[:, 2 * HID:3 * HID])
    keep = _keep_mask(BE)
    c1 = i0 * u0 * keep
    h1 = o0 * jnp.tanh(c1) * keep
    _pack_rec(h1, c1, wfh_ref, rec_ref)


def _phase_c_mid_body(f_ref, wx2_ref, b2_ref, hs_ref, ca_ref, wh_ref, wfh_ref,
                      rec_ref):
    t = (jnp.dot(f_ref[...], wx2_ref[...], preferred_element_type=F32)
         + jnp.dot(hs_ref[...], wh_ref[...], preferred_element_type=F32)
         + b2_ref[...])
    i = _sig(t[:, :HID])
    o = _sig(t[:, HID:2 * HID])
    u = jnp.tanh(t[:, 2 * HID:])
    keep = _keep_mask(BE)
    c = (i * u + ca_ref[...]) * keep
    h = o * jnp.tanh(c) * keep
    _pack_rec(h, c, wfh_ref, rec_ref)


def _phase_c_final_body(f_ref, wx2_ref, b2_ref, hs_ref, ca_ref, wh_ref,
                        h_ref, c_ref):
    t = (jnp.dot(f_ref[...], wx2_ref[...], preferred_element_type=F32)
         + jnp.dot(hs_ref[...], wh_ref[...], preferred_element_type=F32)
         + b2_ref[...])
    i = _sig(t[:, :HID])
    o = _sig(t[:, HID:2 * HID])
    u = jnp.tanh(t[:, 2 * HID:])
    keep = _keep_mask(BE)
    c = (i * u + ca_ref[...]) * keep
    h = o * jnp.tanh(c) * keep
    h_ref[...] = h
    c_ref[...] = c


def _deint(xi):
    ev = plsc.bitcast(lax.shift_left(xi, 16), F32)
    od = plsc.bitcast(lax.bitwise_and(xi, jnp.int32(-65536)), F32)
    return ev, od


def _sc_body(rec_hbm, pref_hbm, idx_hbm, hs_hbm, ca_hbm,
             idx_all, rows0, rows1, pref0, pref1,
             hs0, hs1, ca0, ca1,
             gsem0, gsem1, psem0, psem1, ssem0, ssem1):
    wid = lax.axis_index("s") * NC + lax.axis_index("c")
    base = wid * PER_W
    rows = (rows0, rows1)
    pref = (pref0, pref1)
    hsb = (hs0, hs1)
    cab = (ca0, ca1)
    gsem = (gsem0, gsem1)
    psem = (psem0, psem1)
    ssem = (ssem0, ssem1)

    def g_copy(i, b, n=CH):
        return pltpu.make_async_copy(
            rec_hbm.at[idx_all.at[pl.ds(i * ROWS, n * NNEI)]],
            rows[b].at[pl.ds(0, n * NNEI), :], gsem[b])

    def p_copy(i, b, n=CH):
        return pltpu.make_async_copy(
            pref_hbm.at[pl.ds(base + i * CH, n)], pref[b].at[pl.ds(0, n), :],
            psem[b])

    def s_copies(i, b, n=CH):
        m0 = base + i * CH
        return (pltpu.make_async_copy(hsb[b].at[pl.ds(0, n), :],
                                      hs_hbm.at[pl.ds(m0, n)], ssem[b]),
                pltpu.make_async_copy(cab[b].at[pl.ds(0, n), :],
                                      ca_hbm.at[pl.ds(m0, n)], ssem[b]))

    def compute(b, n=CH):
        @plsc.parallel_loop(0, n, unroll=2)
        def msg_body(m):
            r0 = m * NNEI
            p = [pref[b][m, pl.ds(v * LANES, LANES)] for v in range(NV)]
            hacc = [None] * NV
            cacc = [None] * NV
            for k in range(NNEI):
                r = r0 + k
                for g in range(2):
                    ev, od = _deint(rows[b][r, pl.ds(g * 16, 16)])
                    hacc[2 * g] = ev if k == 0 else hacc[2 * g] + ev
                    hacc[2 * g + 1] = od if k == 0 else hacc[2 * g + 1] + od
                for v in range(NV):
                    gq, cq = _deint(rows[b][r, pl.ds(32 + 16 * v, 16)])
                    f = cq / (1.0 + jnp.exp(p[v] + gq))
                    cacc[v] = f if k == 0 else cacc[v] + f
            for v in range(NV):
                hsb[b][m, pl.ds(v * LANES, LANES)] = hacc[v]
                cab[b][m, pl.ds(v * LANES, LANES)] = cacc[v]

    pltpu.sync_copy(idx_hbm.at[pl.ds(base * NNEI, PER_W * NNEI)], idx_all)
    for b in range(NBUF):
        g_copy(b, b).start()
        p_copy(b, b).start()

    def group_body(pp, carry):
        for b in range(NBUF):
            i = pp * NBUF + b
            g_copy(i, b).wait()
            p_copy(i, b).wait()

            @pl.when(pp > 0)
            def _wait_store():
                sh, sc = s_copies(i, b)
                sh.wait()
                sc.wait()

            compute(b)
            sh, sc = s_copies(i, b)
            sh.start()
            sc.start()

            @pl.when(i + NBUF < NCHUNK)
            def _prefetch():
                g_copy(i + NBUF, b).start()
                p_copy(i + NBUF, b).start()
        return carry

    ngroups = NCHUNK // NBUF
    lax.fori_loop(0, ngroups, group_body, 0)
    t = NCHUNK
    g_copy(t, 0, TAIL).start()
    p_copy(t, 0, TAIL).start()
    g_copy(t, 0, TAIL).wait()
    p_copy(t, 0, TAIL).wait()
    sh, sc = s_copies(NCHUNK - 2, 0)
    sh.wait()
    sc.wait()
    compute(0, TAIL)
    sh, sc = s_copies(t, 0, TAIL)
    sh.start()
    sc.start()
    sh, sc = s_copies(t, 0, TAIL)
    sh.wait()
    sc.wait()
    sh, sc = s_copies(NCHUNK - 1, 1)
    sh.wait()
    sc.wait()


@functools.cache
def _sc_gather():
    mesh = plsc.VectorSubcoreMesh(core_axis_name="c", subcore_axis_name="s")
    return pl.kernel(
        _sc_body,
        mesh=mesh,
        compiler_params=pltpu.CompilerParams(needs_layout_passes=False,
                                             use_tc_tiling_on_sc=True),
        out_type=(jax.ShapeDtypeStruct((E, HID), F32),
                  jax.ShapeDtypeStruct((E, HID), F32)),
        scratch_types=(
            [pltpu.VMEM((PER_W * NNEI,), jnp.int32)]
            + [pltpu.VMEM((ROWS, 128), jnp.int32) for _ in range(NBUF)]
            + [pltpu.VMEM((CH, HID), F32) for _ in range(3 * NBUF)]
            + [pltpu.SemaphoreType.DMA for _ in range(3 * NBUF)]
        ),
    )


def _rows(w):
    return pl.BlockSpec((BE, w), lambda i: (i, 0))


def _full(r, c):
    return pl.BlockSpec((r, c), lambda i: (0, 0))


def kernel(fmess, bgraph, Wi, bi, Wo, bo, Wf, bf, Wu, bu):
    s = jnp.float32(NEG_SCALE)
    Wx = jnp.concatenate(
        [Wi[:, :IN], Wo[:, :IN], Wu[:, :IN], Wf[:, :IN] * s], axis=0).T
    bcat = jnp.concatenate([bi, bo, bu, bf * s]).reshape(1, 4 * HID)
    Wh = jnp.concatenate([Wi[:, IN:], Wo[:, IN:], Wu[:, IN:]], axis=0).T
    q = jnp.concatenate([jnp.arange(0, 16), jnp.arange(32, 48),
                         jnp.arange(16, 32), jnp.arange(48, 64)])
    Wh = Wh[q]
    WfhT = Wf[:, IN:].T * s
    Wx2 = Wx[:, :3 * HID]
    b2 = bcat[:, :3 * HID]
    idx = bgraph.reshape(-1)

    grid = (E // BE,)
    params = pltpu.CompilerParams(dimension_semantics=("parallel",))

    pf, rec = pl.pallas_call(
        _phase_a_body,
        grid=grid,
        in_specs=[_rows(IN), _full(IN, 4 * HID), _full(1, 4 * HID),
                  _full(HID, HID)],
        out_specs=[_rows(HID), _rows(REC)],
        out_shape=[jax.ShapeDtypeStruct((E, HID), F32),
                   jax.ShapeDtypeStruct((E, REC), jnp.int32)],
        compiler_params=params,
    )(fmess, Wx, bcat, WfhT)

    scg = _sc_gather()
    hs, ca = scg(rec, pf, idx)

    rec = pl.pallas_call(
        _phase_c_mid_body,
        grid=grid,
        in_specs=[_rows(IN), _full(IN, 3 * HID), _full(1, 3 * HID),
                  _rows(HID), _rows(HID), _full(HID, 3 * HID), _full(HID, HID)],
        out_specs=_rows(REC),
        out_shape=jax.ShapeDtypeStruct((E, REC), jnp.int32),
        compiler_params=params,
    )(fmess, Wx2, b2, hs, ca, Wh, WfhT)

    hs, ca = scg(rec, pf, idx)

    h, c = pl.pallas_call(
        _phase_c_final_body,
        grid=grid,
        in_specs=[_rows(IN), _full(IN, 3 * HID), _full(1, 3 * HID),
                  _rows(HID), _rows(HID), _full(HID, 3 * HID)],
        out_specs=[_rows(HID), _rows(HID)],
        out_shape=[jax.ShapeDtypeStruct((E, HID), F32),
                   jax.ShapeDtypeStruct((E, HID), F32)],
        compiler_params=params,
    )(fmess, Wx2, b2, hs, ca, Wh)

    return (h, c)


# speedup vs baseline: 1.9992x; 1.0962x over previous
import functools

import jax
import jax.numpy as jnp
from jax import lax
from jax.experimental import pallas as pl
from jax.experimental.pallas import tpu as pltpu
from jax.experimental.pallas import tpu_sc as plsc

E = 160000
NNEI = 8
IN = 128
HID = 64
F32 = jnp.float32
BF16 = jnp.bfloat16
NEG_SCALE = -1.0

BE = 2000
NC = 2
NS = 16
NW = NC * NS
PER_W = E // NW
CH = 16
NCHUNK = PER_W // CH
TAIL = PER_W - NCHUNK * CH
ROWS = CH * NNEI
NBUF = 2
LANES = 16
NV = HID // LANES
REC = 128


def _sig(x):
    return 1.0 / (1.0 + jnp.exp(-x))


def _keep_mask(nrows):
    pid = pl.program_id(0)
    rowids = lax.broadcasted_iota(jnp.int32, (nrows, 1), 0) + pid * nrows
    return (rowids != 0).astype(F32)


def _bf16_word(a, b):
    ai = lax.bitcast_convert_type(a.astype(BF16).astype(F32), jnp.int32)
    bi = lax.bitcast_convert_type(b.astype(BF16).astype(F32), jnp.int32)
    return lax.bitwise_or(lax.shift_right_logical(ai, 16),
                          lax.bitwise_and(bi, jnp.int32(-65536)))


def _pack_rec(h, c, wfh_ref, rec_ref):
    g = jnp.dot(h, wfh_ref[...], preferred_element_type=F32)
    rec_ref[...] = jnp.concatenate(
        [_bf16_word(h[:, :32], h[:, 32:]), _bf16_word(g, c),
         jnp.zeros((BE, 32), jnp.int32)], axis=1)


def _phase_a_body(f_ref, wx_ref, b_ref, wfh_ref, pf_ref, rec_ref):
    pre = jnp.dot(f_ref[...], wx_ref[...], preferred_element_type=F32) + b_ref[...]
    pf_ref[...] = pre[:, 3 * HID:]
    i0 = _sig(pre[:, :HID])
    o0 = _sig(pre[:, HID:2 * HID])
    u0 = jnp.tanh(pre[:, 2 * HID:3 * HID])
    keep = _keep_mask(BE)
    c1 = i0 * u0 * keep
    h1 = o0 * jnp.tanh(c1) * keep
    _pack_rec(h1, c1, wfh_ref, rec_ref)


def _phase_c_mid_body(f_ref, wx2_ref, b2_ref, hs_ref, ca_ref, wh_ref, wfh_ref,
                      rec_ref):
    t = (jnp.dot(f_ref[...], wx2_ref[...], preferred_element_type=F32)
         + jnp.dot(hs_ref[...], wh_ref[...], preferred_element_type=F32)
         + b2_ref[...])
    i = _sig(t[:, :HID])
    o = _sig(t[:, HID:2 * HID])
    u = jnp.tanh(t[:, 2 * HID:])
    keep = _keep_mask(BE)
    c = (i * u + ca_ref[...]) * keep
    h = o * jnp.tanh(c) * keep
    _pack_rec(h, c, wfh_ref, rec_ref)


def _phase_c_final_body(f_ref, wx2_ref, b2_ref, hs_ref, ca_ref, wh_ref,
                        h_ref, c_ref):
    t = (jnp.dot(f_ref[...], wx2_ref[...], preferred_element_type=F32)
         + jnp.dot(hs_ref[...], wh_ref[...], preferred_element_type=F32)
         + b2_ref[...])
    i = _sig(t[:, :HID])
    o = _sig(t[:, HID:2 * HID])
    u = jnp.tanh(t[:, 2 * HID:])
    keep = _keep_mask(BE)
    c = (i * u + ca_ref[...]) * keep
    h = o * jnp.tanh(c) * keep
    h_ref[...] = h
    c_ref[...] = c


def _deint(xi):
    ev = plsc.bitcast(lax.shift_left(xi, 16), F32)
    od = plsc.bitcast(xi, F32)
    return ev, od


def _sc_body(rec_hbm, pref_hbm, idx_hbm, hs_hbm, ca_hbm,
             idx_all, rows0, rows1, pref0, pref1,
             hs0, hs1, ca0, ca1,
             gsem0, gsem1, psem0, psem1, ssem0, ssem1):
    wid = lax.axis_index("s") * NC + lax.axis_index("c")
    base = wid * PER_W
    rows = (rows0, rows1)
    pref = (pref0, pref1)
    hsb = (hs0, hs1)
    cab = (ca0, ca1)
    gsem = (gsem0, gsem1)
    psem = (psem0, psem1)
    ssem = (ssem0, ssem1)

    def g_copy(i, b, n=CH):
        return pltpu.make_async_copy(
            rec_hbm.at[idx_all.at[pl.ds(i * ROWS, n * NNEI)]],
            rows[b].at[pl.ds(0, n * NNEI), :], gsem[b])

    def p_copy(i, b, n=CH):
        return pltpu.make_async_copy(
            pref_hbm.at[pl.ds(base + i * CH, n)], pref[b].at[pl.ds(0, n), :],
            psem[b])

    def s_copies(i, b, n=CH):
        m0 = base + i * CH
        return (pltpu.make_async_copy(hsb[b].at[pl.ds(0, n), :],
                                      hs_hbm.at[pl.ds(m0, n)], ssem[b]),
                pltpu.make_async_copy(cab[b].at[pl.ds(0, n), :],
                                      ca_hbm.at[pl.ds(m0, n)], ssem[b]))

    def compute(b, n=CH):
        @plsc.parallel_loop(0, n, unroll=2)
        def msg_body(m):
            r0 = m * NNEI
            p = [pref[b][m, pl.ds(v * LANES, LANES)] for v in range(NV)]
            hacc = [None] * NV
            cacc = [None] * NV
            for k in range(NNEI):
                r = r0 + k
                for g in range(2):
                    ev, od = _deint(rows[b][r, pl.ds(g * 16, 16)])
                    hacc[2 * g] = ev if k == 0 else hacc[2 * g] + ev
                    hacc[2 * g + 1] = od if k == 0 else hacc[2 * g + 1] + od
                for v in range(NV):
                    gq, cq = _deint(rows[b][r, pl.ds(32 + 16 * v, 16)])
                    f = cq / (1.0 + jnp.exp(p[v] + gq))
                    cacc[v] = f if k == 0 else cacc[v] + f
            for v in range(NV):
                hsb[b][m, pl.ds(v * LANES, LANES)] = hacc[v]
                cab[b][m, pl.ds(v * LANES, LANES)] = cacc[v]

    pltpu.sync_copy(idx_hbm.at[pl.ds(base * NNEI, PER_W * NNEI)], idx_all)
    for b in range(NBUF):
        g_copy(b, b).start()
        p_copy(b, b).start()

    def group_body(pp, carry):
        for b in range(NBUF):
            i = pp * NBUF + b
            g_copy(i, b).wait()
            p_copy(i, b).wait()

            @pl.when(pp > 0)
            def _wait_store():
                sh, sc = s_copies(i, b)
                sh.wait()
                sc.wait()

            compute(b)
            sh, sc = s_copies(i, b)
            sh.start()
            sc.start()

            @pl.when(i + NBUF < NCHUNK)
            def _prefetch():
                g_copy(i + NBUF, b).start()
                p_copy(i + NBUF, b).start()
        return carry

    ngroups = NCHUNK // NBUF
    lax.fori_loop(0, ngroups, group_body, 0)
    t = NCHUNK
    g_copy(t, 0, TAIL).start()
    p_copy(t, 0, TAIL).start()
    g_copy(t, 0, TAIL).wait()
    p_copy(t, 0, TAIL).wait()
    sh, sc = s_copies(NCHUNK - 2, 0)
    sh.wait()
    sc.wait()
    compute(0, TAIL)
    sh, sc = s_copies(t, 0, TAIL)
    sh.start()
    sc.start()
    sh, sc = s_copies(t, 0, TAIL)
    sh.wait()
    sc.wait()
    sh, sc = s_copies(NCHUNK - 1, 1)
    sh.wait()
    sc.wait()


@functools.cache
def _sc_gather():
    mesh = plsc.VectorSubcoreMesh(core_axis_name="c", subcore_axis_name="s")
    return pl.kernel(
        _sc_body,
        mesh=mesh,
        compiler_params=pltpu.CompilerParams(needs_layout_passes=False,
                                             use_tc_tiling_on_sc=True),
        out_type=(jax.ShapeDtypeStruct((E, HID), F32),
                  jax.ShapeDtypeStruct((E, HID), F32)),
        scratch_types=(
            [pltpu.VMEM((PER_W * NNEI,), jnp.int32)]
            + [pltpu.VMEM((ROWS, 128), jnp.int32) for _ in range(NBUF)]
            + [pltpu.VMEM((CH, HID), F32) for _ in range(3 * NBUF)]
            + [pltpu.SemaphoreType.DMA for _ in range(3 * NBUF)]
        ),
    )


def _rows(w):
    return pl.BlockSpec((BE, w), lambda i: (i, 0))


def _full(r, c):
    return pl.BlockSpec((r, c), lambda i: (0, 0))


def kernel(fmess, bgraph, Wi, bi, Wo, bo, Wf, bf, Wu, bu):
    s = jnp.float32(NEG_SCALE)
    Wx = jnp.concatenate(
        [Wi[:, :IN], Wo[:, :IN], Wu[:, :IN], Wf[:, :IN] * s], axis=0).T
    bcat = jnp.concatenate([bi, bo, bu, bf * s]).reshape(1, 4 * HID)
    Wh = jnp.concatenate([Wi[:, IN:], Wo[:, IN:], Wu[:, IN:]], axis=0).T
    q = jnp.concatenate([jnp.arange(0, 16), jnp.arange(32, 48),
                         jnp.arange(16, 32), jnp.arange(48, 64)])
    Wh = Wh[q]
    WfhT = Wf[:, IN:].T * s
    Wx2 = Wx[:, :3 * HID]
    b2 = bcat[:, :3 * HID]
    idx = bgraph.reshape(-1)

    grid = (E // BE,)
    params = pltpu.CompilerParams(dimension_semantics=("parallel",))

    pf, rec = pl.pallas_call(
        _phase_a_body,
        grid=grid,
        in_specs=[_rows(IN), _full(IN, 4 * HID), _full(1, 4 * HID),
                  _full(HID, HID)],
        out_specs=[_rows(HID), _rows(REC)],
        out_shape=[jax.ShapeDtypeStruct((E, HID), F32),
                   jax.ShapeDtypeStruct((E, REC), jnp.int32)],
        compiler_params=params,
    )(fmess, Wx, bcat, WfhT)

    scg = _sc_gather()
    hs, ca = scg(rec, pf, idx)

    rec = pl.pallas_call(
        _phase_c_mid_body,
        grid=grid,
        in_specs=[_rows(IN), _full(IN, 3 * HID), _full(1, 3 * HID),
                  _rows(HID), _rows(HID), _full(HID, 3 * HID), _full(HID, HID)],
        out_specs=_rows(REC),
        out_shape=jax.ShapeDtypeStruct((E, REC), jnp.int32),
        compiler_params=params,
    )(fmess, Wx2, b2, hs, ca, Wh, WfhT)

    hs, ca = scg(rec, pf, idx)

    h, c = pl.pallas_call(
        _phase_c_final_body,
        grid=grid,
        in_specs=[_rows(IN), _full(IN, 3 * HID), _full(1, 3 * HID),
                  _rows(HID), _rows(HID), _full(HID, 3 * HID)],
        out_specs=[_rows(HID), _rows(HID)],
        out_shape=[jax.ShapeDtypeStruct((E, HID), F32),
                   jax.ShapeDtypeStruct((E, HID), F32)],
        compiler_params=params,
    )(fmess, Wx2, b2, hs, ca, Wh)

    return (h, c)
